# Initial kernel scaffold; baseline (speedup 1.0000x reference)
import jax
import jax.numpy as jnp
from jax.experimental import pallas as pl


def kernel(x, edge_index, W1, b1, W2, b2, W3, b3, W4, b4):
    raise NotImplementedError("write your pallas kernel here")


# speedup vs baseline: 11.9280x; 11.9280x over previous
import functools

import jax
import jax.numpy as jnp
from jax import lax
from jax.experimental import pallas as pl
from jax.experimental.pallas import tpu as pltpu
from jax.experimental.pallas import tpu_sc as plsc

N = 10000
NP = 10240
D = 128
E = 320000

NC = 2
NS = 16
NW = NC * NS
EPW = E // NW
CHUNK = 128
NFULL = EPW // CHUNK
REM = EPW - NFULL * CHUNK
ROWS_PER_SUB = NP // NS
ZROWS = 32
DEGW = 128
DINVW = 16

_sc_mesh = plsc.VectorSubcoreMesh(
    core_axis_name="c", subcore_axis_name="s", num_cores=NC, num_subcores=NS
)


def _zero_vmem(buf, rows, width):
    zero = jnp.zeros((16,), jnp.float32)
    for r in range(rows):
        for j in range(width // 16):
            buf[r, pl.ds(j * 16, 16)] = zero


def _make_sc_degree(interpret=False):
    return pl.kernel(
        _sc_degree_body,
        out_type=jax.ShapeDtypeStruct((NC, NP, DEGW), jnp.float32),
        mesh=_sc_mesh,
        scratch_types=[
            pltpu.VMEM((CHUNK,), jnp.int32),
            pltpu.VMEM((REM,), jnp.int32),
            pltpu.VMEM((CHUNK, DEGW), jnp.float32),
            pltpu.VMEM((ZROWS, DEGW), jnp.float32),
            pltpu.VMEM_SHARED((NP, DEGW), jnp.float32),
        ],
        interpret=interpret,
    )


def _sc_degree_body(dst_hbm, out_hbm, idx_v, idx_r, ones_v, zbuf, acc):
    c = lax.axis_index("c")
    s = lax.axis_index("s")
    w = s * NC + c

    _zero_vmem(zbuf, ZROWS, DEGW)
    one = jnp.full((16,), 1.0, jnp.float32)
    for r in range(CHUNK):
        for j in range(DEGW // 16):
            ones_v[r, pl.ds(j * 16, 16)] = one

    def zloop(i, _):
        pltpu.sync_copy(zbuf, acc.at[pl.ds(s * ROWS_PER_SUB + i * ZROWS, ZROWS)])
        return ()

    lax.fori_loop(0, ROWS_PER_SUB // ZROWS, zloop, ())
    plsc.subcore_barrier()

    base0 = w * EPW

    def chunk(i, _):
        pltpu.sync_copy(dst_hbm.at[pl.ds(base0 + i * CHUNK, CHUNK)], idx_v)
        pltpu.sync_copy(ones_v, acc.at[idx_v], add=True)
        return ()

    lax.fori_loop(0, NFULL, chunk, ())
    pltpu.sync_copy(dst_hbm.at[pl.ds(base0 + NFULL * CHUNK, REM)], idx_r)
    pltpu.sync_copy(ones_v.at[pl.ds(0, REM)], acc.at[idx_r], add=True)

    plsc.subcore_barrier()
    pltpu.sync_copy(
        acc.at[pl.ds(s * ROWS_PER_SUB, ROWS_PER_SUB)],
        out_hbm.at[c, pl.ds(s * ROWS_PER_SUB, ROWS_PER_SUB)],
    )


def _make_sc_aggregate(interpret=False):
    return pl.kernel(
        _sc_aggregate_body,
        out_type=jax.ShapeDtypeStruct((NC, NP, D), jnp.float32),
        mesh=_sc_mesh,
        scratch_types=[
            pltpu.VMEM((CHUNK,), jnp.int32),
            pltpu.VMEM((CHUNK,), jnp.int32),
            pltpu.VMEM((REM,), jnp.int32),
            pltpu.VMEM((REM,), jnp.int32),
            pltpu.VMEM((CHUNK, D), jnp.float32),
            pltpu.VMEM((REM, D), jnp.float32),
            pltpu.VMEM((ZROWS, D), jnp.float32),
            pltpu.VMEM_SHARED((NP, D), jnp.float32),
            pltpu.SemaphoreType.DMA,
        ],
        interpret=interpret,
    )


def _sc_aggregate_body(
    hs_hbm, src_hbm, dst_hbm, out_hbm,
    src_v, dst_v, src_r, dst_r, rows_v, rows_r, zbuf, acc, sem,
):
    c = lax.axis_index("c")
    s = lax.axis_index("s")
    w = s * NC + c

    _zero_vmem(zbuf, ZROWS, D)

    def zloop(i, _):
        pltpu.sync_copy(zbuf, acc.at[pl.ds(s * ROWS_PER_SUB + i * ZROWS, ZROWS)])
        return ()

    lax.fori_loop(0, ROWS_PER_SUB // ZROWS, zloop, ())
    plsc.subcore_barrier()

    base0 = w * EPW

    def chunk(i, _):
        b = base0 + i * CHUNK
        pltpu.sync_copy(src_hbm.at[pl.ds(b, CHUNK)], src_v)
        pltpu.async_copy(hs_hbm.at[src_v], rows_v, sem).wait()
        pltpu.sync_copy(dst_hbm.at[pl.ds(b, CHUNK)], dst_v)
        pltpu.sync_copy(rows_v, acc.at[dst_v], add=True)
        return ()

    lax.fori_loop(0, NFULL, chunk, ())
    b = base0 + NFULL * CHUNK
    pltpu.sync_copy(src_hbm.at[pl.ds(b, REM)], src_r)
    pltpu.async_copy(hs_hbm.at[src_r], rows_r, sem).wait()
    pltpu.sync_copy(dst_hbm.at[pl.ds(b, REM)], dst_r)
    pltpu.sync_copy(rows_r, acc.at[dst_r], add=True)

    plsc.subcore_barrier()
    pltpu.sync_copy(
        acc.at[pl.ds(s * ROWS_PER_SUB, ROWS_PER_SUB)],
        out_hbm.at[c, pl.ds(s * ROWS_PER_SUB, ROWS_PER_SUB)],
    )


ROWBLK = 1024
GRID = NP // ROWBLK

_row_spec = pl.BlockSpec((ROWBLK, D), lambda i: (i, 0))
_p_spec = pl.BlockSpec((ROWBLK, DEGW), lambda i: (i, 0))
_dinv_spec = pl.BlockSpec((ROWBLK, DINVW), lambda i: (i, 0))
_w_spec = pl.BlockSpec((D, D), lambda i: (0, 0))
_b_spec = pl.BlockSpec((1, D), lambda i: (0, 0))


def _leaky(y):
    return jnp.where(y >= 0, y, 0.01 * y)


def _tc_first_body(x_ref, w_ref, p0_ref, p1_ref, hs_ref, dinv_ref):
    deg = p0_ref[:, 0:1] + p1_ref[:, 0:1] + 1.0
    dinv = lax.rsqrt(jnp.maximum(deg, 1.0))
    hs_ref[...] = dinv * jnp.dot(
        x_ref[...], w_ref[...], preferred_element_type=jnp.float32
    )
    dinv_ref[...] = jnp.broadcast_to(dinv, (ROWBLK, DINVW))


def _tc_first(x, w1, p0, p1):
    return pl.pallas_call(
        _tc_first_body,
        grid=(GRID,),
        in_specs=[_row_spec, _w_spec, _p_spec, _p_spec],
        out_specs=[_row_spec, _dinv_spec],
        out_shape=[
            jax.ShapeDtypeStruct((NP, D), jnp.float32),
            jax.ShapeDtypeStruct((NP, DINVW), jnp.float32),
        ],
    )(x, w1, p0, p1)


def _tc_mid_body(p0_ref, p1_ref, hs_ref, b_ref, dinv_ref, w_ref, out_ref):
    dinv = dinv_ref[:, 0:1]
    y = dinv * (p0_ref[...] + p1_ref[...] + hs_ref[...]) + b_ref[...]
    xn = _leaky(y)
    out_ref[...] = dinv * jnp.dot(
        xn, w_ref[...], preferred_element_type=jnp.float32
    )


def _tc_mid(p0, p1, hs, b, dinvb, w):
    return pl.pallas_call(
        _tc_mid_body,
        grid=(GRID,),
        in_specs=[_row_spec, _row_spec, _row_spec, _b_spec, _dinv_spec, _w_spec],
        out_specs=_row_spec,
        out_shape=jax.ShapeDtypeStruct((NP, D), jnp.float32),
    )(p0, p1, hs, b, dinvb, w)


def _tc_last_body(p0_ref, p1_ref, hs_ref, b_ref, dinv_ref, out_ref):
    dinv = dinv_ref[:, 0:1]
    y = dinv * (p0_ref[...] + p1_ref[...] + hs_ref[...]) + b_ref[...]
    out_ref[...] = _leaky(y)


def _tc_last(p0, p1, hs, b, dinvb):
    return pl.pallas_call(
        _tc_last_body,
        grid=(GRID,),
        in_specs=[_row_spec, _row_spec, _row_spec, _b_spec, _dinv_spec],
        out_specs=_row_spec,
        out_shape=jax.ShapeDtypeStruct((NP, D), jnp.float32),
    )(p0, p1, hs, b, dinvb)


_sc_degree = _make_sc_degree()
_sc_aggregate = _make_sc_aggregate()


def kernel(x, edge_index, W1, b1, W2, b2, W3, b3, W4, b4):
    src = edge_index[0].astype(jnp.int32)
    dst = edge_index[1].astype(jnp.int32)
    xp = jnp.pad(x, ((0, NP - N), (0, 0)))

    dpart = _sc_degree(dst)
    hs, dinvb = _tc_first(xp, W1, dpart[0], dpart[1])

    for w, b in ((W2, b1), (W3, b2), (W4, b3)):
        p = _sc_aggregate(hs, src, dst)
        hs = _tc_mid(p[0], p[1], hs, b.reshape(1, D), dinvb, w)

    p = _sc_aggregate(hs, src, dst)
    out = _tc_last(p[0], p[1], hs, b4.reshape(1, D), dinvb)
    return out[:N]
